# Initial kernel scaffold; baseline (speedup 1.0000x reference)
import jax
import jax.numpy as jnp
from jax.experimental import pallas as pl


def kernel(x, edge_index, batch, W1, b1, W2, b2, Wfc, bfc):
    raise NotImplementedError("write your pallas kernel here")


# speedup vs baseline: 8.4368x; 8.4368x over previous
import functools

import jax
import jax.numpy as jnp
from jax import lax
from jax.experimental import pallas as pl
from jax.experimental.pallas import tpu as pltpu
from jax.experimental.pallas import tpu_sc as plsc

_N = 10000
_NP = 10240
_E = 640000
_NW = 32
_RPW = 160
_CH = 32
_EP = _NW * _RPW * 128
_NG = 64
_RB = 2048
_F32 = jnp.float32


def _sc_mesh():
    return plsc.VectorSubcoreMesh(core_axis_name="c", subcore_axis_name="s")


def _zero_vmem(buf, rows, cols):
    def row(j, _):
        for l in range(cols // 16):
            buf[j, pl.ds(l * 16, 16)] = jnp.zeros((16,), _F32)
        return 0
    lax.fori_loop(0, rows, row, 0)


def _make_deg_kernel():
    @functools.partial(
        pl.kernel,
        out_type=jax.ShapeDtypeStruct((2, _NP, 128), _F32),
        mesh=_sc_mesh(),
        scratch_types=[
            pltpu.VMEM((_CH, 128), jnp.int32),
            pltpu.VMEM((128, 128), _F32),
            pltpu.VMEM_SHARED((_NP, 128), _F32),
        ],
    )
    def deg_kernel(dst_hbm, out_hbm, didx, ones, acc):
        c = lax.axis_index("c")
        s = lax.axis_index("s")
        w = s * 2 + c

        _zero_vmem(ones, 128, 128)
        for r in range(5):
            pltpu.sync_copy(ones, acc.at[pl.ds(s * 640 + r * 128, 128)])
        plsc.subcore_barrier()

        def fill(j, _):
            for l in range(8):
                ones[j, pl.ds(l * 16, 16)] = jnp.full((16,), 1.0, _F32)
            return 0
        lax.fori_loop(0, 128, fill, 0)

        def chunk(q, _):
            pltpu.sync_copy(dst_hbm.at[w, pl.ds(q * _CH, _CH)], didx)

            def edge_row(j, _):
                pltpu.sync_copy(ones, acc.at[didx.at[j]], add=True)
                return 0
            lax.fori_loop(0, _CH, edge_row, 0)
            return 0
        lax.fori_loop(0, _RPW // _CH, chunk, 0)
        plsc.subcore_barrier()

        pltpu.sync_copy(acc.at[pl.ds(s * 640, 640)],
                        out_hbm.at[c, pl.ds(s * 640, 640)])

    return deg_kernel


def _make_agg_kernel(F):
    @functools.partial(
        pl.kernel,
        out_type=jax.ShapeDtypeStruct((2, _NP, F), _F32),
        mesh=_sc_mesh(),
        scratch_types=[
            pltpu.VMEM((_CH, 128), jnp.int32),
            pltpu.VMEM((_CH, 128), jnp.int32),
            pltpu.VMEM((128, F), _F32),
            pltpu.VMEM_SHARED((_NP, F), _F32),
            pltpu.SemaphoreType.DMA,
        ],
    )
    def agg_kernel(u_hbm, src_hbm, dst_hbm, out_hbm,
                   sidx, didx, rows, acc, sem):
        c = lax.axis_index("c")
        s = lax.axis_index("s")
        w = s * 2 + c

        _zero_vmem(rows, 128, F)
        for r in range(5):
            pltpu.sync_copy(rows, acc.at[pl.ds(s * 640 + r * 128, 128)])
        plsc.subcore_barrier()

        def chunk(q, _):
            pltpu.sync_copy(src_hbm.at[w, pl.ds(q * _CH, _CH)], sidx)
            pltpu.sync_copy(dst_hbm.at[w, pl.ds(q * _CH, _CH)], didx)

            def edge_row(j, _):
                pltpu.async_copy(u_hbm.at[sidx.at[j]], rows, sem).wait()
                pltpu.sync_copy(rows, acc.at[didx.at[j]], add=True)
                return 0
            lax.fori_loop(0, _CH, edge_row, 0)
            return 0
        lax.fori_loop(0, _RPW // _CH, chunk, 0)
        plsc.subcore_barrier()

        pltpu.sync_copy(acc.at[pl.ds(s * 640, 640)],
                        out_hbm.at[c, pl.ds(s * 640, 640)])

    return agg_kernel


_deg_kernel = _make_deg_kernel()
_agg128 = _make_agg_kernel(128)


def _dinv_from(deg_ref):
    d = deg_ref[0] + deg_ref[1]
    dcol = jnp.max(d, axis=1, keepdims=True) + 1.0
    return lax.rsqrt(dcol)


def _tca_body(x_ref, w1_ref, deg_ref, u1_ref):
    dinv = _dinv_from(deg_ref)
    mm = jnp.dot(x_ref[...], w1_ref[...],
                 preferred_element_type=_F32,
                 precision=lax.Precision.HIGHEST)
    u1_ref[...] = mm * dinv


def _tcb_body(g1_ref, u1_ref, deg_ref, w2_ref, b1_ref, u2_ref):
    dinv = _dinv_from(deg_ref)
    h = (g1_ref[0] + g1_ref[1] + u1_ref[...]) * dinv + b1_ref[...]
    h = jnp.maximum(h, 0.0)
    mm = jnp.dot(h, w2_ref[...],
                 preferred_element_type=_F32,
                 precision=lax.Precision.HIGHEST)
    u2_ref[...] = mm * dinv


def _tcc_body(g2_ref, u2_ref, deg_ref, b2_ref, batch_ref, wfc_ref, bfc_ref,
              out_ref, sacc, cacc):
    i = pl.program_id(0)
    dinv = _dinv_from(deg_ref)
    h2 = (g2_ref[0] + g2_ref[1] + u2_ref[...]) * dinv + b2_ref[...]
    h2 = jnp.maximum(h2, 0.0)

    @pl.when(i == 0)
    def _():
        sacc[...] = jnp.zeros_like(sacc)
        cacc[...] = jnp.zeros_like(cacc)

    ps = jnp.zeros((_NG, 128), _F32)
    cs = jnp.zeros((_NG, 128), _F32)
    gids = lax.broadcasted_iota(jnp.int32, (_NG, 128), 0)
    for k in range(_RB // 128):
        bk = batch_ref[k]
        oh = (gids == jnp.broadcast_to(bk, (_NG, 128))).astype(_F32)
        ps = ps + jnp.dot(oh, h2[k * 128:(k + 1) * 128, :],
                          preferred_element_type=_F32,
                          precision=lax.Precision.HIGHEST)
        cs = cs + jnp.sum(oh, axis=1, keepdims=True)
    sacc[...] = sacc[...] + ps
    cacc[...] = cacc[...] + cs

    pooled = sacc[...] / jnp.maximum(cacc[...], 1.0)
    out_ref[...] = jnp.dot(pooled, wfc_ref[...],
                           preferred_element_type=_F32,
                           precision=lax.Precision.HIGHEST) + bfc_ref[...]


def _tca(xp, W1, deg):
    grid = _NP // _RB
    return pl.pallas_call(
        _tca_body,
        grid=(grid,),
        in_specs=[
            pl.BlockSpec((_RB, 128), lambda i: (i, 0)),
            pl.BlockSpec((128, 128), lambda i: (0, 0)),
            pl.BlockSpec((2, _RB, 128), lambda i: (0, i, 0)),
        ],
        out_specs=pl.BlockSpec((_RB, 128), lambda i: (i, 0)),
        out_shape=jax.ShapeDtypeStruct((_NP, 128), _F32),
    )(xp, W1, deg)


def _tcb(g1, u1, deg, W2, b1r):
    grid = _NP // _RB
    return pl.pallas_call(
        _tcb_body,
        grid=(grid,),
        in_specs=[
            pl.BlockSpec((2, _RB, 128), lambda i: (0, i, 0)),
            pl.BlockSpec((_RB, 128), lambda i: (i, 0)),
            pl.BlockSpec((2, _RB, 128), lambda i: (0, i, 0)),
            pl.BlockSpec((128, 128), lambda i: (0, 0)),
            pl.BlockSpec((1, 128), lambda i: (0, 0)),
        ],
        out_specs=pl.BlockSpec((_RB, 128), lambda i: (i, 0)),
        out_shape=jax.ShapeDtypeStruct((_NP, 128), _F32),
    )(g1, u1, deg, W2, b1r)


def _tcc(g2, u2, deg, b2r, batch2, Wfc, bfcr):
    grid = _NP // _RB
    return pl.pallas_call(
        _tcc_body,
        grid=(grid,),
        in_specs=[
            pl.BlockSpec((2, _RB, 128), lambda i: (0, i, 0)),
            pl.BlockSpec((_RB, 128), lambda i: (i, 0)),
            pl.BlockSpec((2, _RB, 128), lambda i: (0, i, 0)),
            pl.BlockSpec((1, 128), lambda i: (0, 0)),
            pl.BlockSpec((_RB // 128, 128), lambda i: (i, 0)),
            pl.BlockSpec((128, 10), lambda i: (0, 0)),
            pl.BlockSpec((1, 10), lambda i: (0, 0)),
        ],
        out_specs=pl.BlockSpec((_NG, 10), lambda i: (0, 0)),
        out_shape=jax.ShapeDtypeStruct((_NG, 10), _F32),
        scratch_shapes=[
            pltpu.VMEM((_NG, 128), _F32),
            pltpu.VMEM((_NG, 128), _F32),
        ],
    )(g2, u2, deg, b2r, batch2, Wfc, bfcr)


def kernel(x, edge_index, batch, W1, b1, W2, b2, Wfc, bfc):
    xp = jnp.zeros((_NP, 128), _F32).at[:_N, :].set(x)
    epad = jnp.full((_EP - _E,), _N, jnp.int32)
    src3 = jnp.concatenate([edge_index[0], epad]).reshape(_NW, _RPW, 128)
    dst3 = jnp.concatenate([edge_index[1], epad]).reshape(_NW, _RPW, 128)
    batch2 = jnp.concatenate(
        [batch, jnp.full((_NP - _N,), _NG, jnp.int32)]).reshape(_NP // 128, 128)
    W1p = jnp.zeros((128, 128), _F32).at[:, :64].set(W1)
    W2p = jnp.zeros((128, 128), _F32).at[:64, :].set(W2)
    b1r = jnp.zeros((1, 128), _F32).at[0, :64].set(b1)
    b2r = b2.reshape(1, 128)
    bfcr = bfc.reshape(1, 10)

    deg = _deg_kernel(dst3)
    u1 = _tca(xp, W1p, deg)
    g1 = _agg128(u1, src3, dst3)
    u2 = _tcb(g1, u1, deg, W2p, b1r)
    g2 = _agg128(u2, src3, dst3)
    return _tcc(g2, u2, deg, b2r, batch2, Wfc, bfcr)


# speedup vs baseline: 9.3444x; 1.1076x over previous
import functools

import jax
import jax.numpy as jnp
from jax import lax
from jax.experimental import pallas as pl
from jax.experimental.pallas import tpu as pltpu
from jax.experimental.pallas import tpu_sc as plsc

_N = 10000
_NP = 10240
_E = 640000
_NW = 32
_RPW = 160
_CH = 32
_EP = _NW * _RPW * 128
_NG = 64
_RB = 2048
_F32 = jnp.float32


def _sc_mesh():
    return plsc.VectorSubcoreMesh(core_axis_name="c", subcore_axis_name="s")


def _zero_vmem(buf, rows, cols):
    def row(j, _):
        for l in range(cols // 16):
            buf[j, pl.ds(l * 16, 16)] = jnp.zeros((16,), _F32)
        return 0
    lax.fori_loop(0, rows, row, 0)


def _make_deg_kernel():
    @functools.partial(
        pl.kernel,
        out_type=jax.ShapeDtypeStruct((2, _NP, 128), _F32),
        mesh=_sc_mesh(),
        scratch_types=[
            pltpu.VMEM((_CH, 128), jnp.int32),
            pltpu.VMEM((128, 128), _F32),
            pltpu.VMEM_SHARED((_NP, 128), _F32),
        ],
    )
    def deg_kernel(dst_hbm, out_hbm, didx, ones, acc):
        c = lax.axis_index("c")
        s = lax.axis_index("s")
        w = s * 2 + c

        _zero_vmem(ones, 128, 128)
        for r in range(5):
            pltpu.sync_copy(ones, acc.at[pl.ds(s * 640 + r * 128, 128)])
        plsc.subcore_barrier()

        def fill(j, _):
            for l in range(8):
                ones[j, pl.ds(l * 16, 16)] = jnp.full((16,), 1.0, _F32)
            return 0
        lax.fori_loop(0, 128, fill, 0)

        def chunk(q, _):
            pltpu.sync_copy(dst_hbm.at[w, pl.ds(q * _CH, _CH)], didx)

            def edge_row(j, _):
                pltpu.sync_copy(ones, acc.at[didx.at[j]], add=True)
                return 0
            lax.fori_loop(0, _CH, edge_row, 0)
            return 0
        lax.fori_loop(0, _RPW // _CH, chunk, 0)
        plsc.subcore_barrier()

        pltpu.sync_copy(acc.at[pl.ds(s * 640, 640)],
                        out_hbm.at[c, pl.ds(s * 640, 640)])

    return deg_kernel


def _make_agg_kernel(F):
    @functools.partial(
        pl.kernel,
        out_type=jax.ShapeDtypeStruct((2, _NP, F), _F32),
        mesh=_sc_mesh(),
        scratch_types=[
            pltpu.VMEM((_CH, 128), jnp.int32),
            pltpu.VMEM((_CH, 128), jnp.int32),
            pltpu.VMEM((2, 128, F), _F32),
            pltpu.VMEM_SHARED((_NP, F), _F32),
            pltpu.SemaphoreType.DMA,
            pltpu.SemaphoreType.DMA,
            pltpu.SemaphoreType.DMA,
            pltpu.SemaphoreType.DMA,
        ],
    )
    def agg_kernel(u_hbm, src_hbm, dst_hbm, out_hbm,
                   sidx, didx, rows, acc, semg0, semg1, sems0, sems1):
        c = lax.axis_index("c")
        s = lax.axis_index("s")
        w = s * 2 + c
        b0 = rows.at[0]
        b1 = rows.at[1]

        def wait_g(slot_ref, idx_ref, sem):
            pltpu.make_async_copy(u_hbm.at[idx_ref], slot_ref, sem).wait()

        def wait_s(slot_ref, idx_ref, sem):
            pltpu.make_async_copy(slot_ref, acc.at[idx_ref], sem).wait()

        _zero_vmem(b0, 128, F)
        for r in range(5):
            pltpu.sync_copy(b0, acc.at[pl.ds(s * 640 + r * 128, 128)])
        plsc.subcore_barrier()

        def chunk(q, _):
            @pl.when(q > 0)
            def _():
                wait_s(b0, didx.at[0], sems0)
                wait_s(b1, didx.at[1], sems1)
            pltpu.sync_copy(src_hbm.at[w, pl.ds(q * _CH, _CH)], sidx)
            pltpu.sync_copy(dst_hbm.at[w, pl.ds(q * _CH, _CH)], didx)
            pltpu.async_copy(u_hbm.at[sidx.at[0]], b0, semg0)

            def pair(t, _):
                j0 = 2 * t
                j1 = j0 + 1

                @pl.when(t > 0)
                def _():
                    wait_s(b1, didx.at[j1], sems1)
                pltpu.async_copy(u_hbm.at[sidx.at[j1]], b1, semg1)
                wait_g(b0, sidx.at[j0], semg0)
                pltpu.async_copy(b0, acc.at[didx.at[j0]], sems0, add=True)

                @pl.when(t < _CH // 2 - 1)
                def _():
                    wait_s(b0, didx.at[j0], sems0)
                    pltpu.async_copy(u_hbm.at[sidx.at[j0 + 2]], b0, semg0)
                wait_g(b1, sidx.at[j1], semg1)
                pltpu.async_copy(b1, acc.at[didx.at[j1]], sems1, add=True)
                return 0
            lax.fori_loop(0, _CH // 2, pair, 0)
            return 0
        lax.fori_loop(0, _RPW // _CH, chunk, 0)
        wait_s(b0, didx.at[0], sems0)
        wait_s(b1, didx.at[1], sems1)
        plsc.subcore_barrier()

        pltpu.sync_copy(acc.at[pl.ds(s * 640, 640)],
                        out_hbm.at[c, pl.ds(s * 640, 640)])

    return agg_kernel


_deg_kernel = _make_deg_kernel()
_agg128 = _make_agg_kernel(128)


def _dinv_from(deg_ref):
    d = deg_ref[0] + deg_ref[1]
    dcol = jnp.max(d, axis=1, keepdims=True) + 1.0
    return lax.rsqrt(dcol)


def _tca_body(x_ref, w1_ref, deg_ref, u1_ref):
    dinv = _dinv_from(deg_ref)
    mm = jnp.dot(x_ref[...], w1_ref[...],
                 preferred_element_type=_F32,
                 precision=lax.Precision.HIGHEST)
    u1_ref[...] = mm * dinv


def _tcb_body(g1_ref, u1_ref, deg_ref, w2_ref, b1_ref, u2_ref):
    dinv = _dinv_from(deg_ref)
    h = (g1_ref[0] + g1_ref[1] + u1_ref[...]) * dinv + b1_ref[...]
    h = jnp.maximum(h, 0.0)
    mm = jnp.dot(h, w2_ref[...],
                 preferred_element_type=_F32,
                 precision=lax.Precision.HIGHEST)
    u2_ref[...] = mm * dinv


def _tcc_body(g2_ref, u2_ref, deg_ref, b2_ref, batch_ref, wfc_ref, bfc_ref,
              out_ref, sacc, cacc):
    i = pl.program_id(0)
    dinv = _dinv_from(deg_ref)
    h2 = (g2_ref[0] + g2_ref[1] + u2_ref[...]) * dinv + b2_ref[...]
    h2 = jnp.maximum(h2, 0.0)

    @pl.when(i == 0)
    def _():
        sacc[...] = jnp.zeros_like(sacc)
        cacc[...] = jnp.zeros_like(cacc)

    ps = jnp.zeros((_NG, 128), _F32)
    cs = jnp.zeros((_NG, 128), _F32)
    gids = lax.broadcasted_iota(jnp.int32, (_NG, 128), 0)
    for k in range(_RB // 128):
        bk = batch_ref[k]
        oh = (gids == jnp.broadcast_to(bk, (_NG, 128))).astype(_F32)
        ps = ps + jnp.dot(oh, h2[k * 128:(k + 1) * 128, :],
                          preferred_element_type=_F32,
                          precision=lax.Precision.HIGHEST)
        cs = cs + jnp.sum(oh, axis=1, keepdims=True)
    sacc[...] = sacc[...] + ps
    cacc[...] = cacc[...] + cs

    pooled = sacc[...] / jnp.maximum(cacc[...], 1.0)
    out_ref[...] = jnp.dot(pooled, wfc_ref[...],
                           preferred_element_type=_F32,
                           precision=lax.Precision.HIGHEST) + bfc_ref[...]


def _tca(xp, W1, deg):
    grid = _NP // _RB
    return pl.pallas_call(
        _tca_body,
        grid=(grid,),
        in_specs=[
            pl.BlockSpec((_RB, 128), lambda i: (i, 0)),
            pl.BlockSpec((128, 128), lambda i: (0, 0)),
            pl.BlockSpec((2, _RB, 128), lambda i: (0, i, 0)),
        ],
        out_specs=pl.BlockSpec((_RB, 128), lambda i: (i, 0)),
        out_shape=jax.ShapeDtypeStruct((_NP, 128), _F32),
    )(xp, W1, deg)


def _tcb(g1, u1, deg, W2, b1r):
    grid = _NP // _RB
    return pl.pallas_call(
        _tcb_body,
        grid=(grid,),
        in_specs=[
            pl.BlockSpec((2, _RB, 128), lambda i: (0, i, 0)),
            pl.BlockSpec((_RB, 128), lambda i: (i, 0)),
            pl.BlockSpec((2, _RB, 128), lambda i: (0, i, 0)),
            pl.BlockSpec((128, 128), lambda i: (0, 0)),
            pl.BlockSpec((1, 128), lambda i: (0, 0)),
        ],
        out_specs=pl.BlockSpec((_RB, 128), lambda i: (i, 0)),
        out_shape=jax.ShapeDtypeStruct((_NP, 128), _F32),
    )(g1, u1, deg, W2, b1r)


def _tcc(g2, u2, deg, b2r, batch2, Wfc, bfcr):
    grid = _NP // _RB
    return pl.pallas_call(
        _tcc_body,
        grid=(grid,),
        in_specs=[
            pl.BlockSpec((2, _RB, 128), lambda i: (0, i, 0)),
            pl.BlockSpec((_RB, 128), lambda i: (i, 0)),
            pl.BlockSpec((2, _RB, 128), lambda i: (0, i, 0)),
            pl.BlockSpec((1, 128), lambda i: (0, 0)),
            pl.BlockSpec((_RB // 128, 128), lambda i: (i, 0)),
            pl.BlockSpec((128, 10), lambda i: (0, 0)),
            pl.BlockSpec((1, 10), lambda i: (0, 0)),
        ],
        out_specs=pl.BlockSpec((_NG, 10), lambda i: (0, 0)),
        out_shape=jax.ShapeDtypeStruct((_NG, 10), _F32),
        scratch_shapes=[
            pltpu.VMEM((_NG, 128), _F32),
            pltpu.VMEM((_NG, 128), _F32),
        ],
    )(g2, u2, deg, b2r, batch2, Wfc, bfcr)


def kernel(x, edge_index, batch, W1, b1, W2, b2, Wfc, bfc):
    xp = jnp.zeros((_NP, 128), _F32).at[:_N, :].set(x)
    epad = jnp.full((_EP - _E,), _N, jnp.int32)
    src3 = jnp.concatenate([edge_index[0], epad]).reshape(_NW, _RPW, 128)
    dst3 = jnp.concatenate([edge_index[1], epad]).reshape(_NW, _RPW, 128)
    batch2 = jnp.concatenate(
        [batch, jnp.full((_NP - _N,), _NG, jnp.int32)]).reshape(_NP // 128, 128)
    W1p = jnp.zeros((128, 128), _F32).at[:, :64].set(W1)
    W2p = jnp.zeros((128, 128), _F32).at[:64, :].set(W2)
    b1r = jnp.zeros((1, 128), _F32).at[0, :64].set(b1)
    b2r = b2.reshape(1, 128)
    bfcr = bfc.reshape(1, 10)

    deg = _deg_kernel(dst3)
    u1 = _tca(xp, W1p, deg)
    g1 = _agg128(u1, src3, dst3)
    u2 = _tcb(g1, u1, deg, W2p, b1r)
    g2 = _agg128(u2, src3, dst3)
    return _tcc(g2, u2, deg, b2r, batch2, Wfc, bfcr)


# speedup vs baseline: 9.3502x; 1.0006x over previous
import functools

import jax
import jax.numpy as jnp
from jax import lax
from jax.experimental import pallas as pl
from jax.experimental.pallas import tpu as pltpu
from jax.experimental.pallas import tpu_sc as plsc

_N = 10000
_NP = 10240
_E = 640000
_NW = 32
_RPW = 160
_CH = 32
_EP = _NW * _RPW * 128
_NG = 64
_RB = 2048
_F32 = jnp.float32


def _sc_mesh():
    return plsc.VectorSubcoreMesh(core_axis_name="c", subcore_axis_name="s")


def _zero_vmem(buf, rows, cols):
    def row(j, _):
        for l in range(cols // 16):
            buf[j, pl.ds(l * 16, 16)] = jnp.zeros((16,), _F32)
        return 0
    lax.fori_loop(0, rows, row, 0)


def _make_deg_kernel():
    @functools.partial(
        pl.kernel,
        out_type=jax.ShapeDtypeStruct((2, _NP, 128), _F32),
        mesh=_sc_mesh(),
        scratch_types=[
            pltpu.VMEM((_CH, 128), jnp.int32),
            pltpu.VMEM((128, 128), _F32),
            pltpu.VMEM_SHARED((_NP, 128), _F32),
        ],
    )
    def deg_kernel(dst_hbm, out_hbm, didx, ones, acc):
        c = lax.axis_index("c")
        s = lax.axis_index("s")
        w = s * 2 + c

        _zero_vmem(ones, 128, 128)
        for r in range(5):
            pltpu.sync_copy(ones, acc.at[pl.ds(s * 640 + r * 128, 128)])
        plsc.subcore_barrier()

        def fill(j, _):
            for l in range(8):
                ones[j, pl.ds(l * 16, 16)] = jnp.full((16,), 1.0, _F32)
            return 0
        lax.fori_loop(0, 128, fill, 0)

        def chunk(q, _):
            pltpu.sync_copy(dst_hbm.at[w, pl.ds(q * _CH, _CH)], didx)

            def edge_row(j, _):
                pltpu.sync_copy(ones, acc.at[didx.at[j]], add=True)
                return 0
            lax.fori_loop(0, _CH, edge_row, 0)
            return 0
        lax.fori_loop(0, _RPW // _CH, chunk, 0)
        plsc.subcore_barrier()

        pltpu.sync_copy(acc.at[pl.ds(s * 640, 640)],
                        out_hbm.at[c, pl.ds(s * 640, 640)])

    return deg_kernel


def _make_agg_kernel(F):
    @functools.partial(
        pl.kernel,
        out_type=jax.ShapeDtypeStruct((2, _NP, F), _F32),
        mesh=_sc_mesh(),
        scratch_types=[
            pltpu.VMEM((_CH, 128), jnp.int32),
            pltpu.VMEM((_CH, 128), jnp.int32),
            pltpu.VMEM((2, 128, F), _F32),
            pltpu.VMEM_SHARED((_NP, F), _F32),
            pltpu.SemaphoreType.DMA,
            pltpu.SemaphoreType.DMA,
            pltpu.SemaphoreType.DMA,
            pltpu.SemaphoreType.DMA,
        ],
    )
    def agg_kernel(u_hbm, src_hbm, dst_hbm, out_hbm,
                   sidx, didx, rows, acc, semg0, semg1, sems0, sems1):
        c = lax.axis_index("c")
        s = lax.axis_index("s")
        w = s * 2 + c
        b0 = rows.at[0]
        b1 = rows.at[1]

        def wait_g(slot_ref, idx_ref, sem):
            pltpu.make_async_copy(u_hbm.at[idx_ref], slot_ref, sem).wait()

        def wait_s(slot_ref, idx_ref, sem):
            pltpu.make_async_copy(slot_ref, acc.at[idx_ref], sem).wait()

        _zero_vmem(b0, 128, F)
        for r in range(5):
            pltpu.sync_copy(b0, acc.at[pl.ds(s * 640 + r * 128, 128)])
        plsc.subcore_barrier()

        def chunk(q, _):
            @pl.when(q > 0)
            def _():
                wait_s(b0, didx.at[0], sems0)
                wait_s(b1, didx.at[1], sems1)
            pltpu.sync_copy(src_hbm.at[w, pl.ds(q * _CH, _CH)], sidx)
            pltpu.sync_copy(dst_hbm.at[w, pl.ds(q * _CH, _CH)], didx)
            pltpu.async_copy(u_hbm.at[sidx.at[0]], b0, semg0)

            def pair(t, _):
                j0 = 2 * t
                j1 = j0 + 1

                @pl.when(t > 0)
                def _():
                    wait_s(b1, didx.at[j1], sems1)
                pltpu.async_copy(u_hbm.at[sidx.at[j1]], b1, semg1)
                wait_g(b0, sidx.at[j0], semg0)
                pltpu.async_copy(b0, acc.at[didx.at[j0]], sems0, add=True)

                @pl.when(t < _CH // 2 - 1)
                def _():
                    wait_s(b0, didx.at[j0], sems0)
                    pltpu.async_copy(u_hbm.at[sidx.at[j0 + 2]], b0, semg0)
                wait_g(b1, sidx.at[j1], semg1)
                pltpu.async_copy(b1, acc.at[didx.at[j1]], sems1, add=True)
                return 0
            lax.fori_loop(0, _CH // 2, pair, 0)
            return 0
        lax.fori_loop(0, _RPW // _CH, chunk, 0)
        wait_s(b0, didx.at[0], sems0)
        wait_s(b1, didx.at[1], sems1)
        plsc.subcore_barrier()

        pltpu.sync_copy(acc.at[pl.ds(s * 640, 640)],
                        out_hbm.at[c, pl.ds(s * 640, 640)])

    return agg_kernel


_deg_kernel = _make_deg_kernel()
_agg128 = _make_agg_kernel(128)


def _dinv_from(deg_ref):
    d = deg_ref[0] + deg_ref[1]
    dcol = jnp.max(d, axis=1, keepdims=True) + 1.0
    return lax.rsqrt(dcol)


def _tca_body(x_ref, w1_ref, deg_ref, u1_ref):
    dinv = _dinv_from(deg_ref)
    mm = jnp.dot(x_ref[...], w1_ref[...],
                 preferred_element_type=_F32,
                 precision=lax.Precision.HIGHEST)
    u1_ref[...] = mm * dinv


def _tcb_body(g1_ref, u1_ref, deg_ref, w2_ref, b1_ref, u2_ref):
    dinv = _dinv_from(deg_ref)
    h = (g1_ref[0] + g1_ref[1] + u1_ref[...]) * dinv + b1_ref[...]
    h = jnp.maximum(h, 0.0)
    mm = jnp.dot(h, w2_ref[...],
                 preferred_element_type=_F32,
                 precision=lax.Precision.HIGHEST)
    u2_ref[...] = mm * dinv


def _tcc_body(g2_ref, u2_ref, deg_ref, b2_ref, batch_ref, wfc_ref, bfc_ref,
              out_ref, sacc, cacc):
    i = pl.program_id(0)
    dinv = _dinv_from(deg_ref)
    h2 = (g2_ref[0] + g2_ref[1] + u2_ref[...]) * dinv + b2_ref[...]
    h2 = jnp.maximum(h2, 0.0)

    @pl.when(i == 0)
    def _():
        sacc[...] = jnp.zeros_like(sacc)
        cacc[...] = jnp.zeros_like(cacc)

    ps = jnp.zeros((_NG, 128), _F32)
    cs = jnp.zeros((_NG, 128), _F32)
    gids = lax.broadcasted_iota(jnp.int32, (_NG, 128), 0)
    for k in range(_RB // 128):
        bk = batch_ref[k]
        oh = (gids == jnp.broadcast_to(bk, (_NG, 128))).astype(_F32)
        ps = ps + jnp.dot(oh, h2[k * 128:(k + 1) * 128, :],
                          preferred_element_type=_F32,
                          precision=lax.Precision.HIGHEST)
        cs = cs + jnp.sum(oh, axis=1, keepdims=True)
    sacc[...] = sacc[...] + ps
    cacc[...] = cacc[...] + cs

    pooled = sacc[...] / jnp.maximum(cacc[...], 1.0)
    out_ref[...] = jnp.dot(pooled, wfc_ref[...],
                           preferred_element_type=_F32,
                           precision=lax.Precision.HIGHEST) + bfc_ref[...]


def _tca(xp, W1, deg):
    grid = _NP // _RB
    return pl.pallas_call(
        _tca_body,
        grid=(grid,),
        in_specs=[
            pl.BlockSpec((_RB, 128), lambda i: (i, 0)),
            pl.BlockSpec((128, 128), lambda i: (0, 0)),
            pl.BlockSpec((2, _RB, 128), lambda i: (0, i, 0)),
        ],
        out_specs=pl.BlockSpec((_RB, 128), lambda i: (i, 0)),
        out_shape=jax.ShapeDtypeStruct((_NP, 128), _F32),
    )(xp, W1, deg)


def _tcb(g1, u1, deg, W2, b1r):
    grid = _NP // _RB
    return pl.pallas_call(
        _tcb_body,
        grid=(grid,),
        in_specs=[
            pl.BlockSpec((2, _RB, 128), lambda i: (0, i, 0)),
            pl.BlockSpec((_RB, 128), lambda i: (i, 0)),
            pl.BlockSpec((2, _RB, 128), lambda i: (0, i, 0)),
            pl.BlockSpec((128, 128), lambda i: (0, 0)),
            pl.BlockSpec((1, 128), lambda i: (0, 0)),
        ],
        out_specs=pl.BlockSpec((_RB, 128), lambda i: (i, 0)),
        out_shape=jax.ShapeDtypeStruct((_NP, 128), _F32),
    )(g1, u1, deg, W2, b1r)


def _tcc(g2, u2, deg, b2r, batch2, Wfc, bfcr):
    grid = _NP // _RB
    return pl.pallas_call(
        _tcc_body,
        grid=(grid,),
        in_specs=[
            pl.BlockSpec((2, _RB, 128), lambda i: (0, i, 0)),
            pl.BlockSpec((_RB, 128), lambda i: (i, 0)),
            pl.BlockSpec((2, _RB, 128), lambda i: (0, i, 0)),
            pl.BlockSpec((1, 128), lambda i: (0, 0)),
            pl.BlockSpec((_RB // 128, 128), lambda i: (i, 0)),
            pl.BlockSpec((128, 10), lambda i: (0, 0)),
            pl.BlockSpec((1, 10), lambda i: (0, 0)),
        ],
        out_specs=pl.BlockSpec((_NG, 10), lambda i: (0, 0)),
        out_shape=jax.ShapeDtypeStruct((_NG, 10), _F32),
        scratch_shapes=[
            pltpu.VMEM((_NG, 128), _F32),
            pltpu.VMEM((_NG, 128), _F32),
        ],
    )(g2, u2, deg, b2r, batch2, Wfc, bfcr)


def kernel(x, edge_index, batch, W1, b1, W2, b2, Wfc, bfc):
    xp = jnp.zeros((_NP, 128), _F32).at[:_N, :].set(x)
    epad_src = jnp.full((_EP - _E,), _N, jnp.int32)
    epad_dst = _N + (jnp.arange(_EP - _E, dtype=jnp.int32) % (_NP - _N))
    src3 = jnp.concatenate([edge_index[0], epad_src]).reshape(_NW, _RPW, 128)
    dst3 = jnp.concatenate([edge_index[1], epad_dst]).reshape(_NW, _RPW, 128)
    batch2 = jnp.concatenate(
        [batch, jnp.full((_NP - _N,), _NG, jnp.int32)]).reshape(_NP // 128, 128)
    W1p = jnp.zeros((128, 128), _F32).at[:, :64].set(W1)
    W2p = jnp.zeros((128, 128), _F32).at[:64, :].set(W2)
    b1r = jnp.zeros((1, 128), _F32).at[0, :64].set(b1)
    b2r = b2.reshape(1, 128)
    bfcr = bfc.reshape(1, 10)

    deg = _deg_kernel(dst3)
    u1 = _tca(xp, W1p, deg)
    g1 = _agg128(u1, src3, dst3)
    u2 = _tcb(g1, u1, deg, W2p, b1r)
    g2 = _agg128(u2, src3, dst3)
    return _tcc(g2, u2, deg, b2r, batch2, Wfc, bfcr)


# speedup vs baseline: 31.8548x; 3.4069x over previous
import functools

import jax
import jax.numpy as jnp
from jax import lax
from jax.experimental import pallas as pl
from jax.experimental.pallas import tpu as pltpu
from jax.experimental.pallas import tpu_sc as plsc

_N = 10000
_NP = 10240
_E = 640000
_NW = 32
_RPW = 160
_CH = 32
_EP = _NW * _RPW * 128
_NG = 64
_RB = 2048
_F32 = jnp.float32


def _sc_mesh():
    return plsc.VectorSubcoreMesh(core_axis_name="c", subcore_axis_name="s")


def _zero_vmem(buf, rows, cols):
    def row(j, _):
        for l in range(cols // 16):
            buf[j, pl.ds(l * 16, 16)] = jnp.zeros((16,), _F32)
        return 0
    lax.fori_loop(0, rows, row, 0)


def _make_deg_kernel():
    @functools.partial(
        pl.kernel,
        out_type=jax.ShapeDtypeStruct((2, _NP, 128), _F32),
        mesh=_sc_mesh(),
        scratch_types=[
            pltpu.VMEM((_CH, 128), jnp.int32),
            pltpu.VMEM((128, 128), _F32),
            pltpu.VMEM_SHARED((_NP, 128), _F32),
        ],
    )
    def deg_kernel(dst_hbm, out_hbm, didx, ones, acc):
        c = lax.axis_index("c")
        s = lax.axis_index("s")
        w = s * 2 + c

        _zero_vmem(ones, 128, 128)
        for r in range(5):
            pltpu.sync_copy(ones, acc.at[pl.ds(s * 640 + r * 128, 128)])
        plsc.subcore_barrier()

        def fill(j, _):
            for l in range(8):
                ones[j, pl.ds(l * 16, 16)] = jnp.full((16,), 1.0, _F32)
            return 0
        lax.fori_loop(0, 128, fill, 0)

        def chunk(q, _):
            pltpu.sync_copy(dst_hbm.at[w, pl.ds(q * _CH, _CH)], didx)

            def edge_row(j, _):
                pltpu.sync_copy(ones, acc.at[didx.at[j]], add=True)
                return 0
            lax.fori_loop(0, _CH, edge_row, 0)
            return 0
        lax.fori_loop(0, _RPW // _CH, chunk, 0)
        plsc.subcore_barrier()

        pltpu.sync_copy(acc.at[pl.ds(s * 640, 640)],
                        out_hbm.at[c, pl.ds(s * 640, 640)])

    return deg_kernel


def _make_agg_kernel(F):
    @functools.partial(
        pl.kernel,
        out_type=jax.ShapeDtypeStruct((2, _NP, F), _F32),
        mesh=_sc_mesh(),
        scratch_types=[
            pltpu.VMEM((_CH, 128), jnp.int32),
            pltpu.VMEM((_CH, 128), jnp.int32),
            pltpu.VMEM((2, 128, F), _F32),
            pltpu.VMEM_SHARED((_NP, F), _F32),
            pltpu.SemaphoreType.DMA,
            pltpu.SemaphoreType.DMA,
            pltpu.SemaphoreType.DMA,
            pltpu.SemaphoreType.DMA,
        ],
    )
    def agg_kernel(u_hbm, src_hbm, dst_hbm, out_hbm,
                   sidx, didx, rows, acc, semg0, semg1, sems0, sems1):
        c = lax.axis_index("c")
        s = lax.axis_index("s")
        w = s * 2 + c
        b0 = rows.at[0]
        b1 = rows.at[1]

        def wait_g(slot_ref, idx_ref, sem):
            pltpu.make_async_copy(u_hbm.at[idx_ref], slot_ref, sem).wait()

        def wait_s(slot_ref, idx_ref, sem):
            pltpu.make_async_copy(slot_ref, acc.at[idx_ref], sem).wait()

        _zero_vmem(b0, 128, F)
        for r in range(5):
            pltpu.sync_copy(b0, acc.at[pl.ds(s * 640 + r * 128, 128)])
        plsc.subcore_barrier()

        def chunk(q, _):
            @pl.when(q > 0)
            def _():
                wait_s(b0, didx.at[0], sems0)
                wait_s(b1, didx.at[1], sems1)
            pltpu.sync_copy(src_hbm.at[w, pl.ds(q * _CH, _CH)], sidx)
            pltpu.sync_copy(dst_hbm.at[w, pl.ds(q * _CH, _CH)], didx)
            pltpu.async_copy(u_hbm.at[sidx.at[0]], b0, semg0)

            def pair(t, _):
                j0 = 2 * t
                j1 = j0 + 1

                @pl.when(t > 0)
                def _():
                    wait_s(b1, didx.at[j1], sems1)
                pltpu.async_copy(u_hbm.at[sidx.at[j1]], b1, semg1)
                wait_g(b0, sidx.at[j0], semg0)
                pltpu.async_copy(b0, acc.at[didx.at[j0]], sems0, add=True)

                @pl.when(t < _CH // 2 - 1)
                def _():
                    wait_s(b0, didx.at[j0], sems0)
                    pltpu.async_copy(u_hbm.at[sidx.at[j0 + 2]], b0, semg0)
                wait_g(b1, sidx.at[j1], semg1)
                pltpu.async_copy(b1, acc.at[didx.at[j1]], sems1, add=True)
                return 0
            lax.fori_loop(0, _CH // 2, pair, 0)
            return 0
        lax.fori_loop(0, _RPW // _CH, chunk, 0)
        wait_s(b0, didx.at[0], sems0)
        wait_s(b1, didx.at[1], sems1)
        plsc.subcore_barrier()

        pltpu.sync_copy(acc.at[pl.ds(s * 640, 640)],
                        out_hbm.at[c, pl.ds(s * 640, 640)])

    return agg_kernel


_deg_kernel = _make_deg_kernel()
_agg128 = _make_agg_kernel(128)


def _dinv_from(deg_ref):
    d = deg_ref[0] + deg_ref[1]
    dcol = jnp.max(d, axis=1, keepdims=True) + 1.0
    return lax.rsqrt(dcol)


def _tca_body(x_ref, w1_ref, deg_ref, u1_ref):
    dinv = _dinv_from(deg_ref)
    mm = jnp.dot(x_ref[...], w1_ref[...],
                 preferred_element_type=_F32,
                 precision=lax.Precision.HIGHEST)
    u1_ref[...] = mm * dinv


def _tcb_body(g1_ref, u1_ref, deg_ref, w2_ref, b1_ref, u2_ref):
    dinv = _dinv_from(deg_ref)
    h = (g1_ref[0] + g1_ref[1] + u1_ref[...]) * dinv + b1_ref[...]
    h = jnp.maximum(h, 0.0)
    mm = jnp.dot(h, w2_ref[...],
                 preferred_element_type=_F32,
                 precision=lax.Precision.HIGHEST)
    u2_ref[...] = mm * dinv


def _tcc_body(g2_ref, u2_ref, deg_ref, b2_ref, batch_ref, wfc_ref, bfc_ref,
              out_ref, sacc, cacc):
    i = pl.program_id(0)
    dinv = _dinv_from(deg_ref)
    h2 = (g2_ref[0] + g2_ref[1] + u2_ref[...]) * dinv + b2_ref[...]
    h2 = jnp.maximum(h2, 0.0)

    @pl.when(i == 0)
    def _():
        sacc[...] = jnp.zeros_like(sacc)
        cacc[...] = jnp.zeros_like(cacc)

    ps = jnp.zeros((_NG, 128), _F32)
    cs = jnp.zeros((_NG, 128), _F32)
    gids = lax.broadcasted_iota(jnp.int32, (_NG, 128), 0)
    for k in range(_RB // 128):
        bk = batch_ref[k]
        oh = (gids == jnp.broadcast_to(bk, (_NG, 128))).astype(_F32)
        ps = ps + jnp.dot(oh, h2[k * 128:(k + 1) * 128, :],
                          preferred_element_type=_F32,
                          precision=lax.Precision.HIGHEST)
        cs = cs + jnp.sum(oh, axis=1, keepdims=True)
    sacc[...] = sacc[...] + ps
    cacc[...] = cacc[...] + cs

    pooled = sacc[...] / jnp.maximum(cacc[...], 1.0)
    out_ref[...] = jnp.dot(pooled, wfc_ref[...],
                           preferred_element_type=_F32,
                           precision=lax.Precision.HIGHEST) + bfc_ref[...]


def _tca(xp, W1, deg):
    grid = _NP // _RB
    return pl.pallas_call(
        _tca_body,
        grid=(grid,),
        in_specs=[
            pl.BlockSpec((_RB, 128), lambda i: (i, 0)),
            pl.BlockSpec((128, 128), lambda i: (0, 0)),
            pl.BlockSpec((2, _RB, 128), lambda i: (0, i, 0)),
        ],
        out_specs=pl.BlockSpec((_RB, 128), lambda i: (i, 0)),
        out_shape=jax.ShapeDtypeStruct((_NP, 128), _F32),
    )(xp, W1, deg)


def _tcb(g1, u1, deg, W2, b1r):
    grid = _NP // _RB
    return pl.pallas_call(
        _tcb_body,
        grid=(grid,),
        in_specs=[
            pl.BlockSpec((2, _RB, 128), lambda i: (0, i, 0)),
            pl.BlockSpec((_RB, 128), lambda i: (i, 0)),
            pl.BlockSpec((2, _RB, 128), lambda i: (0, i, 0)),
            pl.BlockSpec((128, 128), lambda i: (0, 0)),
            pl.BlockSpec((1, 128), lambda i: (0, 0)),
        ],
        out_specs=pl.BlockSpec((_RB, 128), lambda i: (i, 0)),
        out_shape=jax.ShapeDtypeStruct((_NP, 128), _F32),
    )(g1, u1, deg, W2, b1r)


def _tcc(g2, u2, deg, b2r, batch2, Wfc, bfcr):
    grid = _NP // _RB
    return pl.pallas_call(
        _tcc_body,
        grid=(grid,),
        in_specs=[
            pl.BlockSpec((2, _RB, 128), lambda i: (0, i, 0)),
            pl.BlockSpec((_RB, 128), lambda i: (i, 0)),
            pl.BlockSpec((2, _RB, 128), lambda i: (0, i, 0)),
            pl.BlockSpec((1, 128), lambda i: (0, 0)),
            pl.BlockSpec((_RB // 128, 128), lambda i: (i, 0)),
            pl.BlockSpec((128, 10), lambda i: (0, 0)),
            pl.BlockSpec((1, 10), lambda i: (0, 0)),
        ],
        out_specs=pl.BlockSpec((_NG, 10), lambda i: (0, 0)),
        out_shape=jax.ShapeDtypeStruct((_NG, 10), _F32),
        scratch_shapes=[
            pltpu.VMEM((_NG, 128), _F32),
            pltpu.VMEM((_NG, 128), _F32),
        ],
    )(g2, u2, deg, b2r, batch2, Wfc, bfcr)


def kernel(x, edge_index, batch, W1, b1, W2, b2, Wfc, bfc):
    xp = jnp.zeros((_NP, 128), _F32).at[:_N, :].set(x)
    epad_src = _N + (jnp.arange(_EP - _E, dtype=jnp.int32) % (_NP - _N))
    epad_dst = epad_src
    src3 = jnp.concatenate([edge_index[0], epad_src]).reshape(_NW, _RPW, 128)
    dst3 = jnp.concatenate([edge_index[1], epad_dst]).reshape(_NW, _RPW, 128)
    batch2 = jnp.concatenate(
        [batch, jnp.full((_NP - _N,), _NG, jnp.int32)]).reshape(_NP // 128, 128)
    W1p = jnp.zeros((128, 128), _F32).at[:, :64].set(W1)
    W2p = jnp.zeros((128, 128), _F32).at[:64, :].set(W2)
    b1r = jnp.zeros((1, 128), _F32).at[0, :64].set(b1)
    b2r = b2.reshape(1, 128)
    bfcr = bfc.reshape(1, 10)

    deg = _deg_kernel(dst3)
    u1 = _tca(xp, W1p, deg)
    g1 = _agg128(u1, src3, dst3)
    u2 = _tcb(g1, u1, deg, W2p, b1r)
    g2 = _agg128(u2, src3, dst3)
    return _tcc(g2, u2, deg, b2r, batch2, Wfc, bfcr)


# speedup vs baseline: 38.2753x; 1.2016x over previous
import functools

import jax
import jax.numpy as jnp
from jax import lax
from jax.experimental import pallas as pl
from jax.experimental.pallas import tpu as pltpu
from jax.experimental.pallas import tpu_sc as plsc

_N = 10000
_NP = 10240
_E = 640000
_NW = 32
_RPW = 160
_CH = 32
_EP = _NW * _RPW * 128
_NG = 64
_RB = 2048
_F32 = jnp.float32


def _sc_mesh():
    return plsc.VectorSubcoreMesh(core_axis_name="c", subcore_axis_name="s")


def _zero_vmem(buf, rows, cols):
    def row(j, _):
        for l in range(cols // 16):
            buf[j, pl.ds(l * 16, 16)] = jnp.zeros((16,), _F32)
        return 0
    lax.fori_loop(0, rows, row, 0)


def _make_deg_kernel():
    seg = _NP // 16

    @functools.partial(
        pl.kernel,
        out_type=jax.ShapeDtypeStruct((2, _NP, 128), _F32),
        mesh=_sc_mesh(),
        compiler_params=pltpu.CompilerParams(needs_layout_passes=False),
        scratch_types=[
            pltpu.VMEM((_RPW, 128), jnp.int32),
            pltpu.VMEM((_NP,), _F32),
            pltpu.VMEM((seg,), _F32),
            pltpu.VMEM((seg,), _F32),
            pltpu.VMEM((seg, 128), _F32),
            pltpu.VMEM_SHARED((16, _NP), _F32),
        ],
    )
    def deg_kernel(dst_hbm, out_hbm, didx, hist, ta, tb, outblk, parts):
        c = lax.axis_index("c")
        s = lax.axis_index("s")
        w = s * 2 + c

        pltpu.sync_copy(dst_hbm.at[w], didx)

        def zero(i, _):
            hist[pl.ds(i * 16, 16)] = jnp.zeros((16,), _F32)
            return 0
        lax.fori_loop(0, _NP // 16, zero, 0)

        ones16 = jnp.full((16,), 1.0, _F32)

        def edge_row(j, _):
            for l in range(8):
                idx = didx[j, pl.ds(l * 16, 16)]
                plsc.addupdate_scatter(hist, [idx], ones16)
            return 0
        lax.fori_loop(0, _RPW, edge_row, 0)

        pltpu.sync_copy(hist, parts.at[s])
        plsc.subcore_barrier()

        pltpu.sync_copy(parts.at[0, pl.ds(s * seg, seg)], ta)
        for p in range(1, 16):
            pltpu.sync_copy(parts.at[p, pl.ds(s * seg, seg)], tb)

            def madd(i, _):
                ta[pl.ds(i * 16, 16)] = (ta[pl.ds(i * 16, 16)] +
                                         tb[pl.ds(i * 16, 16)])
                return 0
            lax.fori_loop(0, seg // 16, madd, 0)

        zcol = jnp.zeros((16,), jnp.int32)
        base_rows = lax.iota(jnp.int32, 16)

        def col_write(i, _):
            vals = ta[pl.ds(i * 16, 16)]
            plsc.store_scatter(outblk, [base_rows + i * 16, zcol], vals)
            return 0
        lax.fori_loop(0, seg // 16, col_write, 0)

        pltpu.sync_copy(outblk, out_hbm.at[c, pl.ds(s * seg, seg)])

    return deg_kernel


def _make_agg_kernel(F):
    @functools.partial(
        pl.kernel,
        out_type=jax.ShapeDtypeStruct((2, _NP, F), _F32),
        mesh=_sc_mesh(),
        scratch_types=[
            pltpu.VMEM((_CH, 128), jnp.int32),
            pltpu.VMEM((_CH, 128), jnp.int32),
            pltpu.VMEM((2, 128, F), _F32),
            pltpu.VMEM_SHARED((_NP, F), _F32),
            pltpu.SemaphoreType.DMA,
            pltpu.SemaphoreType.DMA,
            pltpu.SemaphoreType.DMA,
            pltpu.SemaphoreType.DMA,
        ],
    )
    def agg_kernel(u_hbm, src_hbm, dst_hbm, out_hbm,
                   sidx, didx, rows, acc, semg0, semg1, sems0, sems1):
        c = lax.axis_index("c")
        s = lax.axis_index("s")
        w = s * 2 + c
        b0 = rows.at[0]
        b1 = rows.at[1]

        def wait_g(slot_ref, idx_ref, sem):
            pltpu.make_async_copy(u_hbm.at[idx_ref], slot_ref, sem).wait()

        def wait_s(slot_ref, idx_ref, sem):
            pltpu.make_async_copy(slot_ref, acc.at[idx_ref], sem).wait()

        _zero_vmem(b0, 128, F)
        for r in range(5):
            pltpu.sync_copy(b0, acc.at[pl.ds(s * 640 + r * 128, 128)])
        plsc.subcore_barrier()

        def chunk(q, _):
            @pl.when(q > 0)
            def _():
                wait_s(b0, didx.at[0], sems0)
                wait_s(b1, didx.at[1], sems1)
            pltpu.sync_copy(src_hbm.at[w, pl.ds(q * _CH, _CH)], sidx)
            pltpu.sync_copy(dst_hbm.at[w, pl.ds(q * _CH, _CH)], didx)
            pltpu.async_copy(u_hbm.at[sidx.at[0]], b0, semg0)

            def pair(t, _):
                j0 = 2 * t
                j1 = j0 + 1

                @pl.when(t > 0)
                def _():
                    wait_s(b1, didx.at[j1], sems1)
                pltpu.async_copy(u_hbm.at[sidx.at[j1]], b1, semg1)
                wait_g(b0, sidx.at[j0], semg0)
                pltpu.async_copy(b0, acc.at[didx.at[j0]], sems0, add=True)

                @pl.when(t < _CH // 2 - 1)
                def _():
                    wait_s(b0, didx.at[j0], sems0)
                    pltpu.async_copy(u_hbm.at[sidx.at[j0 + 2]], b0, semg0)
                wait_g(b1, sidx.at[j1], semg1)
                pltpu.async_copy(b1, acc.at[didx.at[j1]], sems1, add=True)
                return 0
            lax.fori_loop(0, _CH // 2, pair, 0)
            return 0
        lax.fori_loop(0, _RPW // _CH, chunk, 0)
        wait_s(b0, didx.at[0], sems0)
        wait_s(b1, didx.at[1], sems1)
        plsc.subcore_barrier()

        pltpu.sync_copy(acc.at[pl.ds(s * 640, 640)],
                        out_hbm.at[c, pl.ds(s * 640, 640)])

    return agg_kernel


_deg_kernel = _make_deg_kernel()
_agg128 = _make_agg_kernel(128)


def _dinv_from(deg_ref):
    dcol = deg_ref[0][:, 0:1] + deg_ref[1][:, 0:1] + 1.0
    return lax.rsqrt(dcol)


def _tca_body(x_ref, w1_ref, deg_ref, u1_ref):
    dinv = _dinv_from(deg_ref)
    mm = jnp.dot(x_ref[...], w1_ref[...],
                 preferred_element_type=_F32,
                 precision=lax.Precision.HIGHEST)
    u1_ref[...] = mm * dinv


def _tcb_body(g1_ref, u1_ref, deg_ref, w2_ref, b1_ref, u2_ref):
    dinv = _dinv_from(deg_ref)
    h = (g1_ref[0] + g1_ref[1] + u1_ref[...]) * dinv + b1_ref[...]
    h = jnp.maximum(h, 0.0)
    mm = jnp.dot(h, w2_ref[...],
                 preferred_element_type=_F32,
                 precision=lax.Precision.HIGHEST)
    u2_ref[...] = mm * dinv


def _tcc_body(g2_ref, u2_ref, deg_ref, b2_ref, batch_ref, wfc_ref, bfc_ref,
              out_ref, sacc, cacc):
    i = pl.program_id(0)
    dinv = _dinv_from(deg_ref)
    h2 = (g2_ref[0] + g2_ref[1] + u2_ref[...]) * dinv + b2_ref[...]
    h2 = jnp.maximum(h2, 0.0)

    @pl.when(i == 0)
    def _():
        sacc[...] = jnp.zeros_like(sacc)
        cacc[...] = jnp.zeros_like(cacc)

    ps = jnp.zeros((_NG, 128), _F32)
    cs = jnp.zeros((_NG, 128), _F32)
    gids = lax.broadcasted_iota(jnp.int32, (_NG, 128), 0)
    for k in range(_RB // 128):
        bk = batch_ref[k]
        oh = (gids == jnp.broadcast_to(bk, (_NG, 128))).astype(_F32)
        ps = ps + jnp.dot(oh, h2[k * 128:(k + 1) * 128, :],
                          preferred_element_type=_F32,
                          precision=lax.Precision.HIGHEST)
        cs = cs + jnp.sum(oh, axis=1, keepdims=True)
    sacc[...] = sacc[...] + ps
    cacc[...] = cacc[...] + cs

    pooled = sacc[...] / jnp.maximum(cacc[...], 1.0)
    out_ref[...] = jnp.dot(pooled, wfc_ref[...],
                           preferred_element_type=_F32,
                           precision=lax.Precision.HIGHEST) + bfc_ref[...]


def _tca(xp, W1, deg):
    grid = _NP // _RB
    return pl.pallas_call(
        _tca_body,
        grid=(grid,),
        in_specs=[
            pl.BlockSpec((_RB, 128), lambda i: (i, 0)),
            pl.BlockSpec((128, 128), lambda i: (0, 0)),
            pl.BlockSpec((2, _RB, 128), lambda i: (0, i, 0)),
        ],
        out_specs=pl.BlockSpec((_RB, 128), lambda i: (i, 0)),
        out_shape=jax.ShapeDtypeStruct((_NP, 128), _F32),
    )(xp, W1, deg)


def _tcb(g1, u1, deg, W2, b1r):
    grid = _NP // _RB
    return pl.pallas_call(
        _tcb_body,
        grid=(grid,),
        in_specs=[
            pl.BlockSpec((2, _RB, 128), lambda i: (0, i, 0)),
            pl.BlockSpec((_RB, 128), lambda i: (i, 0)),
            pl.BlockSpec((2, _RB, 128), lambda i: (0, i, 0)),
            pl.BlockSpec((128, 128), lambda i: (0, 0)),
            pl.BlockSpec((1, 128), lambda i: (0, 0)),
        ],
        out_specs=pl.BlockSpec((_RB, 128), lambda i: (i, 0)),
        out_shape=jax.ShapeDtypeStruct((_NP, 128), _F32),
    )(g1, u1, deg, W2, b1r)


def _tcc(g2, u2, deg, b2r, batch2, Wfc, bfcr):
    grid = _NP // _RB
    return pl.pallas_call(
        _tcc_body,
        grid=(grid,),
        in_specs=[
            pl.BlockSpec((2, _RB, 128), lambda i: (0, i, 0)),
            pl.BlockSpec((_RB, 128), lambda i: (i, 0)),
            pl.BlockSpec((2, _RB, 128), lambda i: (0, i, 0)),
            pl.BlockSpec((1, 128), lambda i: (0, 0)),
            pl.BlockSpec((_RB // 128, 128), lambda i: (i, 0)),
            pl.BlockSpec((128, 10), lambda i: (0, 0)),
            pl.BlockSpec((1, 10), lambda i: (0, 0)),
        ],
        out_specs=pl.BlockSpec((_NG, 10), lambda i: (0, 0)),
        out_shape=jax.ShapeDtypeStruct((_NG, 10), _F32),
        scratch_shapes=[
            pltpu.VMEM((_NG, 128), _F32),
            pltpu.VMEM((_NG, 128), _F32),
        ],
    )(g2, u2, deg, b2r, batch2, Wfc, bfcr)


def kernel(x, edge_index, batch, W1, b1, W2, b2, Wfc, bfc):
    xp = jnp.zeros((_NP, 128), _F32).at[:_N, :].set(x)
    epad_src = _N + (jnp.arange(_EP - _E, dtype=jnp.int32) % (_NP - _N))
    epad_dst = epad_src
    src3 = jnp.concatenate([edge_index[0], epad_src]).reshape(_NW, _RPW, 128)
    dst3 = jnp.concatenate([edge_index[1], epad_dst]).reshape(_NW, _RPW, 128)
    batch2 = jnp.concatenate(
        [batch, jnp.full((_NP - _N,), _NG, jnp.int32)]).reshape(_NP // 128, 128)
    W1p = jnp.zeros((128, 128), _F32).at[:, :64].set(W1)
    W2p = jnp.zeros((128, 128), _F32).at[:64, :].set(W2)
    b1r = jnp.zeros((1, 128), _F32).at[0, :64].set(b1)
    b2r = b2.reshape(1, 128)
    bfcr = bfc.reshape(1, 10)

    deg = _deg_kernel(dst3)
    u1 = _tca(xp, W1p, deg)
    g1 = _agg128(u1, src3, dst3)
    u2 = _tcb(g1, u1, deg, W2p, b1r)
    g2 = _agg128(u2, src3, dst3)
    return _tcc(g2, u2, deg, b2r, batch2, Wfc, bfcr)


# speedup vs baseline: 38.7703x; 1.0129x over previous
import functools

import jax
import jax.numpy as jnp
from jax import lax
from jax.experimental import pallas as pl
from jax.experimental.pallas import tpu as pltpu
from jax.experimental.pallas import tpu_sc as plsc

_N = 10000
_NP = 10240
_E = 640000
_NW = 32
_RPW = 160
_CH = 40
_EP = _NW * _RPW * 128
_NG = 64
_RB = 2048
_F32 = jnp.float32


def _sc_mesh():
    return plsc.VectorSubcoreMesh(core_axis_name="c", subcore_axis_name="s")


def _zero_vmem(buf, rows, cols):
    def row(j, _):
        for l in range(cols // 16):
            buf[j, pl.ds(l * 16, 16)] = jnp.zeros((16,), _F32)
        return 0
    lax.fori_loop(0, rows, row, 0)


def _make_deg_kernel():
    seg = _NP // 16

    @functools.partial(
        pl.kernel,
        out_type=jax.ShapeDtypeStruct((2, _NP, 128), _F32),
        mesh=_sc_mesh(),
        compiler_params=pltpu.CompilerParams(needs_layout_passes=False),
        scratch_types=[
            pltpu.VMEM((_RPW, 128), jnp.int32),
            pltpu.VMEM((_NP,), _F32),
            pltpu.VMEM((seg,), _F32),
            pltpu.VMEM((seg,), _F32),
            pltpu.VMEM((seg, 128), _F32),
            pltpu.VMEM_SHARED((16, _NP), _F32),
        ],
    )
    def deg_kernel(dst_hbm, out_hbm, didx, hist, ta, tb, outblk, parts):
        c = lax.axis_index("c")
        s = lax.axis_index("s")
        w = s * 2 + c

        pltpu.sync_copy(dst_hbm.at[w], didx)

        def zero(i, _):
            hist[pl.ds(i * 16, 16)] = jnp.zeros((16,), _F32)
            return 0
        lax.fori_loop(0, _NP // 16, zero, 0)

        ones16 = jnp.full((16,), 1.0, _F32)

        def edge_row(j, _):
            for l in range(8):
                idx = didx[j, pl.ds(l * 16, 16)]
                plsc.addupdate_scatter(hist, [idx], ones16)
            return 0
        lax.fori_loop(0, _RPW, edge_row, 0)

        pltpu.sync_copy(hist, parts.at[s])
        plsc.subcore_barrier()

        pltpu.sync_copy(parts.at[0, pl.ds(s * seg, seg)], ta)
        for p in range(1, 16):
            pltpu.sync_copy(parts.at[p, pl.ds(s * seg, seg)], tb)

            def madd(i, _):
                ta[pl.ds(i * 16, 16)] = (ta[pl.ds(i * 16, 16)] +
                                         tb[pl.ds(i * 16, 16)])
                return 0
            lax.fori_loop(0, seg // 16, madd, 0)

        zcol = jnp.zeros((16,), jnp.int32)
        base_rows = lax.iota(jnp.int32, 16)

        def col_write(i, _):
            vals = ta[pl.ds(i * 16, 16)]
            plsc.store_scatter(outblk, [base_rows + i * 16, zcol], vals)
            return 0
        lax.fori_loop(0, seg // 16, col_write, 0)

        pltpu.sync_copy(outblk, out_hbm.at[c, pl.ds(s * seg, seg)])

    return deg_kernel


def _make_agg_kernel(F):
    @functools.partial(
        pl.kernel,
        out_type=jax.ShapeDtypeStruct((2, _NP, F), _F32),
        mesh=_sc_mesh(),
        scratch_types=[
            pltpu.VMEM((_CH, 128), jnp.int32),
            pltpu.VMEM((_CH, 128), jnp.int32),
            pltpu.VMEM((2, 128, F), _F32),
            pltpu.VMEM_SHARED((_NP, F), _F32),
            pltpu.SemaphoreType.DMA,
            pltpu.SemaphoreType.DMA,
            pltpu.SemaphoreType.DMA,
            pltpu.SemaphoreType.DMA,
        ],
    )
    def agg_kernel(u_hbm, src_hbm, dst_hbm, out_hbm,
                   sidx, didx, rows, acc, semg0, semg1, sems0, sems1):
        c = lax.axis_index("c")
        s = lax.axis_index("s")
        w = s * 2 + c
        b0 = rows.at[0]
        b1 = rows.at[1]

        def wait_g(slot_ref, idx_ref, sem):
            pltpu.make_async_copy(u_hbm.at[idx_ref], slot_ref, sem).wait()

        def wait_s(slot_ref, idx_ref, sem):
            pltpu.make_async_copy(slot_ref, acc.at[idx_ref], sem).wait()

        _zero_vmem(b0, 128, F)
        for r in range(5):
            pltpu.sync_copy(b0, acc.at[pl.ds(s * 640 + r * 128, 128)])
        plsc.subcore_barrier()

        def chunk(q, _):
            @pl.when(q > 0)
            def _():
                wait_s(b0, didx.at[0], sems0)
                wait_s(b1, didx.at[1], sems1)
            pltpu.sync_copy(src_hbm.at[w, pl.ds(q * _CH, _CH)], sidx)
            pltpu.sync_copy(dst_hbm.at[w, pl.ds(q * _CH, _CH)], didx)
            pltpu.async_copy(u_hbm.at[sidx.at[0]], b0, semg0)

            def pair(t, _):
                j0 = 2 * t
                j1 = j0 + 1

                @pl.when(t > 0)
                def _():
                    wait_s(b1, didx.at[j1], sems1)
                pltpu.async_copy(u_hbm.at[sidx.at[j1]], b1, semg1)
                wait_g(b0, sidx.at[j0], semg0)
                pltpu.async_copy(b0, acc.at[didx.at[j0]], sems0, add=True)

                @pl.when(t < _CH // 2 - 1)
                def _():
                    wait_s(b0, didx.at[j0], sems0)
                    pltpu.async_copy(u_hbm.at[sidx.at[j0 + 2]], b0, semg0)
                wait_g(b1, sidx.at[j1], semg1)
                pltpu.async_copy(b1, acc.at[didx.at[j1]], sems1, add=True)
                return 0
            lax.fori_loop(0, _CH // 2, pair, 0)
            return 0
        lax.fori_loop(0, _RPW // _CH, chunk, 0)
        wait_s(b0, didx.at[0], sems0)
        wait_s(b1, didx.at[1], sems1)
        plsc.subcore_barrier()

        pltpu.sync_copy(acc.at[pl.ds(s * 640, 640)],
                        out_hbm.at[c, pl.ds(s * 640, 640)])

    return agg_kernel


_deg_kernel = _make_deg_kernel()
_agg128 = _make_agg_kernel(128)


def _dinv_from(deg_ref):
    dcol = deg_ref[0][:, 0:1] + deg_ref[1][:, 0:1] + 1.0
    return lax.rsqrt(dcol)


def _tca_body(x_ref, w1_ref, deg_ref, u1_ref):
    dinv = _dinv_from(deg_ref)
    mm = jnp.dot(x_ref[...], w1_ref[...],
                 preferred_element_type=_F32,
                 precision=lax.Precision.HIGHEST)
    u1_ref[...] = mm * dinv


def _tcb_body(g1_ref, u1_ref, deg_ref, w2_ref, b1_ref, u2_ref):
    dinv = _dinv_from(deg_ref)
    h = (g1_ref[0] + g1_ref[1] + u1_ref[...]) * dinv + b1_ref[...]
    h = jnp.maximum(h, 0.0)
    mm = jnp.dot(h, w2_ref[...],
                 preferred_element_type=_F32,
                 precision=lax.Precision.HIGHEST)
    u2_ref[...] = mm * dinv


def _tcc_body(g2_ref, u2_ref, deg_ref, b2_ref, batch_ref, wfc_ref, bfc_ref,
              out_ref, sacc, cacc):
    i = pl.program_id(0)
    dinv = _dinv_from(deg_ref)
    h2 = (g2_ref[0] + g2_ref[1] + u2_ref[...]) * dinv + b2_ref[...]
    h2 = jnp.maximum(h2, 0.0)

    @pl.when(i == 0)
    def _():
        sacc[...] = jnp.zeros_like(sacc)
        cacc[...] = jnp.zeros_like(cacc)

    ps = jnp.zeros((_NG, 128), _F32)
    cs = jnp.zeros((_NG, 128), _F32)
    gids = lax.broadcasted_iota(jnp.int32, (_NG, 128), 0)
    for k in range(_RB // 128):
        bk = batch_ref[k]
        oh = (gids == jnp.broadcast_to(bk, (_NG, 128))).astype(_F32)
        ps = ps + jnp.dot(oh, h2[k * 128:(k + 1) * 128, :],
                          preferred_element_type=_F32,
                          precision=lax.Precision.HIGHEST)
        cs = cs + jnp.sum(oh, axis=1, keepdims=True)
    sacc[...] = sacc[...] + ps
    cacc[...] = cacc[...] + cs

    pooled = sacc[...] / jnp.maximum(cacc[...], 1.0)
    out_ref[...] = jnp.dot(pooled, wfc_ref[...],
                           preferred_element_type=_F32,
                           precision=lax.Precision.HIGHEST) + bfc_ref[...]


def _tca(xp, W1, deg):
    grid = _NP // _RB
    return pl.pallas_call(
        _tca_body,
        grid=(grid,),
        in_specs=[
            pl.BlockSpec((_RB, 128), lambda i: (i, 0)),
            pl.BlockSpec((128, 128), lambda i: (0, 0)),
            pl.BlockSpec((2, _RB, 128), lambda i: (0, i, 0)),
        ],
        out_specs=pl.BlockSpec((_RB, 128), lambda i: (i, 0)),
        out_shape=jax.ShapeDtypeStruct((_NP, 128), _F32),
    )(xp, W1, deg)


def _tcb(g1, u1, deg, W2, b1r):
    grid = _NP // _RB
    return pl.pallas_call(
        _tcb_body,
        grid=(grid,),
        in_specs=[
            pl.BlockSpec((2, _RB, 128), lambda i: (0, i, 0)),
            pl.BlockSpec((_RB, 128), lambda i: (i, 0)),
            pl.BlockSpec((2, _RB, 128), lambda i: (0, i, 0)),
            pl.BlockSpec((128, 128), lambda i: (0, 0)),
            pl.BlockSpec((1, 128), lambda i: (0, 0)),
        ],
        out_specs=pl.BlockSpec((_RB, 128), lambda i: (i, 0)),
        out_shape=jax.ShapeDtypeStruct((_NP, 128), _F32),
    )(g1, u1, deg, W2, b1r)


def _tcc(g2, u2, deg, b2r, batch2, Wfc, bfcr):
    grid = _NP // _RB
    return pl.pallas_call(
        _tcc_body,
        grid=(grid,),
        in_specs=[
            pl.BlockSpec((2, _RB, 128), lambda i: (0, i, 0)),
            pl.BlockSpec((_RB, 128), lambda i: (i, 0)),
            pl.BlockSpec((2, _RB, 128), lambda i: (0, i, 0)),
            pl.BlockSpec((1, 128), lambda i: (0, 0)),
            pl.BlockSpec((_RB // 128, 128), lambda i: (i, 0)),
            pl.BlockSpec((128, 10), lambda i: (0, 0)),
            pl.BlockSpec((1, 10), lambda i: (0, 0)),
        ],
        out_specs=pl.BlockSpec((_NG, 10), lambda i: (0, 0)),
        out_shape=jax.ShapeDtypeStruct((_NG, 10), _F32),
        scratch_shapes=[
            pltpu.VMEM((_NG, 128), _F32),
            pltpu.VMEM((_NG, 128), _F32),
        ],
    )(g2, u2, deg, b2r, batch2, Wfc, bfcr)


def kernel(x, edge_index, batch, W1, b1, W2, b2, Wfc, bfc):
    xp = jnp.zeros((_NP, 128), _F32).at[:_N, :].set(x)
    epad_src = _N + (jnp.arange(_EP - _E, dtype=jnp.int32) % (_NP - _N))
    epad_dst = epad_src
    src3 = jnp.concatenate([edge_index[0], epad_src]).reshape(_NW, _RPW, 128)
    dst3 = jnp.concatenate([edge_index[1], epad_dst]).reshape(_NW, _RPW, 128)
    batch2 = jnp.concatenate(
        [batch, jnp.full((_NP - _N,), _NG, jnp.int32)]).reshape(_NP // 128, 128)
    W1p = jnp.zeros((128, 128), _F32).at[:, :64].set(W1)
    W2p = jnp.zeros((128, 128), _F32).at[:64, :].set(W2)
    b1r = jnp.zeros((1, 128), _F32).at[0, :64].set(b1)
    b2r = b2.reshape(1, 128)
    bfcr = bfc.reshape(1, 10)

    deg = _deg_kernel(dst3)
    u1 = _tca(xp, W1p, deg)
    g1 = _agg128(u1, src3, dst3)
    u2 = _tcb(g1, u1, deg, W2p, b1r)
    g2 = _agg128(u2, src3, dst3)
    return _tcc(g2, u2, deg, b2r, batch2, Wfc, bfcr)
